# Initial kernel scaffold; baseline (speedup 1.0000x reference)
import functools

import jax
import jax.numpy as jnp
from jax import lax
from jax.experimental import pallas as pl
from jax.experimental.pallas import tpu as pltpu
from jax.experimental.pallas import tpu_sc as plsc

ROWS, COLS, EMBED = 16384, 26, 32
B = ROWS * COLS
NW = 32
B_PER_W = B // NW
IDX_MINOR = 128
CH_J = 8
CHUNK = CH_J * IDX_MINOR
N_CHUNK = B_PER_W // CHUNK
ROWS_PER_W = B_PER_W // IDX_MINOR

_mesh = plsc.VectorSubcoreMesh(core_axis_name="c", subcore_axis_name="s")


@functools.partial(
    pl.kernel,
    mesh=_mesh,
    out_type=jax.ShapeDtypeStruct((B, EMBED), jnp.float32),
    scratch_types=[
        pltpu.VMEM((CH_J, IDX_MINOR), jnp.int32),
        pltpu.VMEM((CHUNK, EMBED), jnp.float32),
        pltpu.SemaphoreType.DMA,
    ],
)
def _embed_lookup(idx_hbm, table_hbm, out_hbm, idx_v, rows_v, sem):
    wid = lax.axis_index("s") * 2 + lax.axis_index("c")
    row_base = wid * ROWS_PER_W

    def chunk_body(i, _):
        row_off = row_base + i * CH_J
        flat_off = row_off * IDX_MINOR
        pltpu.sync_copy(idx_hbm.at[pl.ds(row_off, CH_J)], idx_v)
        copies = []
        for j in range(CH_J):
            copies.append(
                pltpu.async_copy(
                    table_hbm.at[idx_v.at[j]],
                    rows_v.at[pl.ds(j * IDX_MINOR, IDX_MINOR)],
                    sem,
                )
            )
        for c in copies:
            c.wait()
        pltpu.sync_copy(rows_v, out_hbm.at[pl.ds(flat_off, CHUNK)])
        return 0

    lax.fori_loop(0, N_CHUNK, chunk_body, 0)


def kernel(ms, table):
    idx2d = ms.reshape(B // IDX_MINOR, IDX_MINOR)
    out = _embed_lookup(idx2d, table)
    return out.reshape(ROWS, COLS, EMBED)


# speedup vs baseline: 1.5479x; 1.5479x over previous
import functools

import jax
import jax.numpy as jnp
from jax import lax
from jax.experimental import pallas as pl
from jax.experimental.pallas import tpu as pltpu
from jax.experimental.pallas import tpu_sc as plsc

ROWS, COLS, EMBED = 16384, 26, 32
B = ROWS * COLS
NW = 32
B_PER_W = B // NW
IDX_MINOR = 128
CH_J = 8
CHUNK = CH_J * IDX_MINOR
N_CHUNK = B_PER_W // CHUNK
ROWS_PER_W = B_PER_W // IDX_MINOR

_mesh = plsc.VectorSubcoreMesh(core_axis_name="c", subcore_axis_name="s")


@functools.partial(
    pl.kernel,
    mesh=_mesh,
    out_type=jax.ShapeDtypeStruct((B, EMBED), jnp.float32),
    scratch_types=[
        pltpu.VMEM((CH_J, IDX_MINOR), jnp.int32),
        pltpu.VMEM((CHUNK, EMBED), jnp.float32),
        pltpu.SemaphoreType.DMA,
    ],
    compiler_params=pltpu.CompilerParams(use_tc_tiling_on_sc=False),
)
def _embed_lookup(idx_hbm, table_hbm, out_hbm, idx_v, rows_v, sem):
    wid = lax.axis_index("s") * 2 + lax.axis_index("c")
    row_base = wid * ROWS_PER_W

    def chunk_body(i, _):
        row_off = row_base + i * CH_J
        flat_off = row_off * IDX_MINOR
        pltpu.sync_copy(idx_hbm.at[pl.ds(row_off, CH_J)], idx_v)
        copies = []
        for j in range(CH_J):
            copies.append(
                pltpu.async_copy(
                    table_hbm.at[idx_v.at[j]],
                    rows_v.at[pl.ds(j * IDX_MINOR, IDX_MINOR)],
                    sem,
                )
            )
        for c in copies:
            c.wait()
        pltpu.sync_copy(rows_v, out_hbm.at[pl.ds(flat_off, CHUNK)])
        return 0

    lax.fori_loop(0, N_CHUNK, chunk_body, 0)


def kernel(ms, table):
    idx2d = ms.reshape(B // IDX_MINOR, IDX_MINOR)
    out = _embed_lookup(idx2d, table)
    return out.reshape(ROWS, COLS, EMBED)


# speedup vs baseline: 1.5677x; 1.0127x over previous
import functools

import jax
import jax.numpy as jnp
from jax import lax
from jax.experimental import pallas as pl
from jax.experimental.pallas import tpu as pltpu
from jax.experimental.pallas import tpu_sc as plsc

ROWS, COLS, EMBED = 16384, 26, 32
B = ROWS * COLS
NW = 32
B_PER_W = B // NW
IDX_MINOR = 128
CH_J = 13
CHUNK = CH_J * IDX_MINOR
N_CHUNK = B_PER_W // CHUNK
ROWS_PER_W = B_PER_W // IDX_MINOR

_mesh = plsc.VectorSubcoreMesh(core_axis_name="c", subcore_axis_name="s")


@functools.partial(
    pl.kernel,
    mesh=_mesh,
    out_type=jax.ShapeDtypeStruct((B, EMBED), jnp.float32),
    scratch_types=[
        pltpu.VMEM((2, CH_J, IDX_MINOR), jnp.int32),
        pltpu.VMEM((2, CHUNK, EMBED), jnp.float32),
        pltpu.SemaphoreType.DMA,
        pltpu.SemaphoreType.DMA,
        pltpu.SemaphoreType.DMA,
    ],
    compiler_params=pltpu.CompilerParams(use_tc_tiling_on_sc=False),
)
def _embed_lookup(idx_hbm, table_hbm, out_hbm, idx_v, rows_v, gsem, osem0,
                  osem1):
    wid = lax.axis_index("s") * 2 + lax.axis_index("c")
    row_base = wid * ROWS_PER_W
    osems = (osem0, osem1)

    def do_chunk(c, b, wait_prev_store):
        row_off = row_base + c * CH_J
        flat_off = row_off * IDX_MINOR
        my_idx = idx_v.at[b]
        my_rows = rows_v.at[b]
        pltpu.sync_copy(idx_hbm.at[pl.ds(row_off, CH_J)], my_idx)
        if wait_prev_store:
            pltpu.make_async_copy(
                my_rows, out_hbm.at[pl.ds(flat_off, CHUNK)], osems[b]
            ).wait()
        copies = [
            pltpu.async_copy(
                table_hbm.at[my_idx.at[j]],
                my_rows.at[pl.ds(j * IDX_MINOR, IDX_MINOR)],
                gsem,
            )
            for j in range(CH_J)
        ]
        for cp in copies:
            cp.wait()
        pltpu.async_copy(my_rows, out_hbm.at[pl.ds(flat_off, CHUNK)],
                         osems[b])

    do_chunk(0, 0, False)
    do_chunk(1, 1, False)

    def pair_body(p, _):
        do_chunk(2 * p, 0, True)
        do_chunk(2 * p + 1, 1, True)
        return 0

    lax.fori_loop(1, N_CHUNK // 2, pair_body, 0)

    for b in range(2):
        pltpu.make_async_copy(
            rows_v.at[b], out_hbm.at[pl.ds(0, CHUNK)], osems[b]
        ).wait()


def kernel(ms, table):
    idx2d = ms.reshape(B // IDX_MINOR, IDX_MINOR)
    out = _embed_lookup(idx2d, table)
    return out.reshape(ROWS, COLS, EMBED)
